# baseline (speedup 1.0000x reference)
import functools

import jax
import jax.numpy as jnp
from jax import lax
from jax.experimental import pallas as pl
from jax.experimental.pallas import tpu as pltpu
from jax.experimental.pallas import tpu_sc as plsc

NSC, NSUB = 2, 16
NWORK = NSC * NSUB
F32 = jnp.float32
XW = 256


def _sc_gather_rows(table, idx2):
    R, D = table.shape
    G, C = idx2.shape
    E = G * C
    iters = G // NWORK
    mesh = plsc.VectorSubcoreMesh(core_axis_name="c", subcore_axis_name="s")

    @functools.partial(
        pl.kernel,
        out_type=jax.ShapeDtypeStruct((E, D), F32),
        mesh=mesh,
        scratch_types=[
            pltpu.VMEM((iters, C), jnp.int32),
            pltpu.VMEM((C, D), F32),
            pltpu.VMEM((C, D), F32),
            pltpu.SemaphoreType.DMA,
            pltpu.SemaphoreType.DMA,
            pltpu.SemaphoreType.DMA,
            pltpu.SemaphoreType.DMA,
        ],
    )
    def gk(table_hbm, idx_hbm, out_hbm, idx_v, r0, r1, sg0, sg1, ss0, ss1):
        wid = lax.axis_index("s") * NSC + lax.axis_index("c")
        g0 = wid * iters
        pltpu.sync_copy(idx_hbm.at[pl.ds(g0, iters)], idx_v)
        rows = (r0, r1)
        semg = (sg0, sg1)
        sems = (ss0, ss1)
        hg = [None, None]
        hs = [None, None]
        for i in range(iters):
            b = i & 1
            if i >= 2:
                hs[b].wait()
            hg[b] = pltpu.async_copy(table_hbm.at[idx_v.at[i]], rows[b],
                                     semg[b])
            if i >= 1:
                p = 1 - b
                hg[p].wait()
                hs[p] = pltpu.async_copy(
                    rows[p], out_hbm.at[pl.ds((g0 + i - 1) * C, C)], sems[p])
        last = iters - 1
        hg[last & 1].wait()
        hs[last & 1] = pltpu.async_copy(
            rows[last & 1], out_hbm.at[pl.ds((g0 + last) * C, C)],
            sems[last & 1])
        if iters >= 2:
            hs[(last - 1) & 1].wait()
        hs[last & 1].wait()

    return gk(table, idx2)


def _topk_body(caf_ref, cat_ref, src_ref, *, L, RB, K):
    b = pl.program_id(0)
    r = pl.program_id(1)
    a = caf_ref[0]
    bt = cat_ref[0]
    d = jnp.zeros((RB, L), F32)
    for c in range(3):
        dc = a[:, c:c + 1] - bt[c:c + 1, :]
        d = d + dc * dc
    jcol = lax.broadcasted_iota(jnp.int32, (RB, L), 1).astype(F32)
    irow = (r * RB + lax.broadcasted_iota(jnp.int32, (RB, L), 0)).astype(F32)
    d = jnp.where(jcol == irow, 1e18, d)
    off = b * L
    for k in range(K):
        m = jnp.min(d, axis=1, keepdims=True)
        eq = d == m
        idxf = jnp.min(jnp.where(eq, jcol, 3e4), axis=1, keepdims=True)
        src_ref[0, :, k:k + 1] = idxf.astype(jnp.int32) + off
        d = jnp.where(eq, 1e17, d)


def _topk(caf_pad, cat_pad, L, K):
    B = caf_pad.shape[0]
    RB = 256
    return pl.pallas_call(
        functools.partial(_topk_body, L=L, RB=RB, K=K),
        grid=(B, L // RB),
        in_specs=[
            pl.BlockSpec((1, RB, 16), lambda b, r: (b, r, 0)),
            pl.BlockSpec((1, 16, L), lambda b, r: (b, 0, 0)),
        ],
        out_specs=pl.BlockSpec((1, RB, K), lambda b, r: (b, r, 0)),
        out_shape=jax.ShapeDtypeStruct((B, L, K), jnp.int32),
    )(caf_pad, cat_pad)


def _node_embed_body(dih_ref, ori_ref, caf_ref, Wh, Ws, bs, Wv, Wg, bg,
                     x_ref, *, NB):
    dih = dih_ref[...]
    ori = ori_ref[...]
    Vh = [jnp.dot(ori[:, 2 * c:2 * c + 2], Wh[...],
                  preferred_element_type=F32) for c in range(3)]
    vn = jnp.sqrt(Vh[0] * Vh[0] + Vh[1] * Vh[1] + Vh[2] * Vh[2] + 1e-8)
    sm = (jnp.dot(dih[:, 0:6], Ws[0:6, :], preferred_element_type=F32)
          + jnp.dot(vn, Ws[6:22, :], preferred_element_type=F32) + bs[...])
    gate = jax.nn.sigmoid(jnp.dot(sm, Wg[...], preferred_element_type=F32)
                          + bg[...])
    Vo = [jnp.dot(h, Wv[...], preferred_element_type=F32) * gate for h in Vh]
    pad = jnp.zeros((NB, XW - 192), F32)
    x_ref[...] = jnp.concatenate([sm] + Vo + [caf_ref[...], pad], axis=1)


def _node_embed(dih8, ori8, caf_flat, p):
    N = dih8.shape[0]
    NB = 256
    ws = (p["Wh"], p["Ws"], p["bs2"], p["Wv"], p["Wg"], p["bg2"])
    wspecs = [pl.BlockSpec(w.shape, lambda i: (0,) * w.ndim) for w in ws]
    return pl.pallas_call(
        functools.partial(_node_embed_body, NB=NB),
        grid=(N // NB,),
        in_specs=[pl.BlockSpec((NB, 8), lambda i: (i, 0)),
                  pl.BlockSpec((NB, 8), lambda i: (i, 0)),
                  pl.BlockSpec((NB, 16), lambda i: (i, 0))] + wspecs,
        out_specs=pl.BlockSpec((NB, XW), lambda i: (i, 0)),
        out_shape=jax.ShapeDtypeStruct((N, XW), F32),
    )(dih8, ori8, caf_flat, *ws)


def _gvp_packed(s, V48, Whb, Wse, bse, Wvb, act):
    Vh = jnp.dot(V48, Whb[...], preferred_element_type=F32)
    vn = jnp.sqrt(Vh[:, 0:16] * Vh[:, 0:16] + Vh[:, 16:32] * Vh[:, 16:32]
                  + Vh[:, 32:48] * Vh[:, 32:48] + 1e-8)
    smq = jnp.dot(jnp.concatenate([s, vn], axis=1), Wse[...],
                  preferred_element_type=F32) + bse[...]
    gate = jax.nn.sigmoid(smq[:, 128:144])
    g3 = jnp.concatenate([gate, gate, gate], axis=1)
    Vo = jnp.dot(Vh, Wvb[...], preferred_element_type=F32) * g3
    so = jax.nn.relu(smq[:, 0:128]) if act else smq[:, 0:128]
    return so, Vo


def _ln(s, g, b):
    m = jnp.mean(s, axis=1, keepdims=True)
    v = jnp.mean((s - m) * (s - m), axis=1, keepdims=True)
    return (s - m) / jnp.sqrt(v + 1e-5) * g[...] + b[...]


def _vln48(V48):
    n = jnp.sqrt(jnp.sum(V48 * V48, axis=1, keepdims=True) / 16.0 + 1e-4)
    return V48 / n


def _edge_gvp(Xg, caf_blk, cen, eWh, eWs, ebs, eWv, eWgT, ebg, *, NB):
    ne = NB * 16
    ca_d = jnp.broadcast_to(caf_blk[:, None, :], (NB, 16, 16)).reshape(ne, 16)
    dv = Xg[:, 176:192] - ca_d
    dist = jnp.sqrt(jnp.sum(dv * dv, axis=1, keepdims=True) + 1e-8)
    rbf = jnp.exp(-(((dist - cen[...]) / 1.25) ** 2))
    Vh = (dv / dist) * eWh[0, 0]
    vn = jnp.sqrt(jnp.sum(Vh * Vh, axis=1, keepdims=True) + 1e-8)
    sm = (jnp.dot(rbf, eWs[0:16, :], preferred_element_type=F32)
          + vn * eWs[16:17, :] + ebs[...])
    gate = jax.nn.sigmoid(
        jnp.sum(sm * eWgT[...], axis=1, keepdims=True) + ebg[...])
    eV16 = Vh * (eWv[0, 0] * gate)
    return sm, eV16


def _layer_body(first, NB, x_ref, xg_ref, *refs):
    if first:
        (caf_b, cen, eWh, eWs, ebs, eWv, eWg, ebg) = refs[:8]
        refs = refs[8:]
    else:
        ef_ref = refs[0]
        refs = refs[1:]
    (Wn0, bn0, We0, Wvn0, Wv0p,
     Whb1, Ws1, bs1, Wvb1,
     Whb2, Ws2, bs2, Wvb2,
     g1, b1,
     Whf0, Wsf0, bsf0, Wvf0,
     Whf1, Wsf1, bsf1, Wvf1,
     g2, b2) = refs[:25]
    outs = refs[25:]
    x_out = outs[0]
    ne = NB * 16
    X = x_ref[...]
    Xg = xg_ref[...]
    if first:
        es, eV16 = _edge_gvp(Xg, caf_b[...], cen, eWh, eWs, ebs, eWv, eWg,
                             ebg, NB=NB)
        efv = jnp.concatenate([es, eV16], axis=1)
        outs[1][...] = efv
    else:
        efv = ef_ref[...]
    s_d = X[:, 0:128]
    Vd48 = X[:, 128:176]

    def rep(t):
        return jnp.broadcast_to(t[:, None, :], (NB, 16, t.shape[1])
                                ).reshape(ne, t.shape[1])

    Mn = jnp.dot(X[:, 0:176], Wn0[...], preferred_element_type=F32) + bn0[...]
    ein = jnp.concatenate([Xg[:, 0:176], efv], axis=1)
    M = jnp.dot(ein, We0[...], preferred_element_type=F32) + rep(Mn)
    Vh = M[:, 144:243]
    vn = jnp.sqrt(Vh[:, 0:33] * Vh[:, 0:33] + Vh[:, 33:66] * Vh[:, 33:66]
                  + Vh[:, 66:99] * Vh[:, 66:99] + 1e-8)
    smq = (M[:, 0:144]
           + jnp.dot(vn, Wvn0[...], preferred_element_type=F32))
    gate = jax.nn.sigmoid(smq[:, 128:144])
    ms = jax.nn.relu(smq[:, 0:128])
    mV = (jnp.dot(Vh, Wv0p[...], preferred_element_type=F32)
          * jnp.concatenate([gate, gate, gate], axis=1))
    ms, mV = _gvp_packed(ms, mV, Whb1, Ws1, bs1, Wvb1, True)
    ms, mV = _gvp_packed(ms, mV, Whb2, Ws2, bs2, Wvb2, False)
    ags = jnp.sum(ms.reshape(NB, 16, 128), axis=1) * (1.0 / 16.0)
    agV = jnp.sum(mV.reshape(NB, 16, 48), axis=1) * (1.0 / 16.0)
    s_n = _ln(s_d + ags, g1, b1)
    V_n = _vln48(Vd48 + agV)
    fs, fV = _gvp_packed(s_n, V_n, Whf0, Wsf0, bsf0, Wvf0, True)
    fs, fV = _gvp_packed(fs, fV, Whf1, Wsf1, bsf1, Wvf1, False)
    s_o = _ln(s_n + fs, g2, b2)
    V_o = _vln48(V_n + fV)
    x_out[...] = jnp.concatenate([s_o, V_o, X[:, 176:XW]], axis=1)


def _layer(X, Xg, wlist, extra):
    N = X.shape[0]
    E = Xg.shape[0]
    NB = 256
    first = len(extra) > 1
    if first:
        especs = ([pl.BlockSpec((NB, 16), lambda i: (i, 0)),
                   pl.BlockSpec((1, 16), lambda i: (0, 0))]
                  + [pl.BlockSpec(w.shape, lambda i: (0,) * w.ndim)
                     for w in extra[2:]])
        out_specs = [pl.BlockSpec((NB, XW), lambda i: (i, 0)),
                     pl.BlockSpec((NB * 16, 48), lambda i: (i, 0))]
        out_shape = [jax.ShapeDtypeStruct((N, XW), F32),
                     jax.ShapeDtypeStruct((E, 48), F32)]
    else:
        especs = [pl.BlockSpec((NB * 16, 48), lambda i: (i, 0))]
        out_specs = [pl.BlockSpec((NB, XW), lambda i: (i, 0))]
        out_shape = [jax.ShapeDtypeStruct((N, XW), F32)]
    wspecs = [pl.BlockSpec(w.shape, lambda i: (0,) * w.ndim) for w in wlist]
    res = pl.pallas_call(
        functools.partial(_layer_body, first, NB),
        grid=(N // NB,),
        in_specs=[pl.BlockSpec((NB, XW), lambda i: (i, 0)),
                  pl.BlockSpec((NB * 16, XW), lambda i: (i, 0))]
        + especs + wspecs,
        out_specs=out_specs,
        out_shape=out_shape,
    )(X, Xg, *extra, *wlist)
    return res if first else (res[0], None)


def _unit(v, axis=-1, eps=1e-8):
    return v / jnp.sqrt(jnp.sum(v * v, axis=axis, keepdims=True) + eps)


def _dih_feats(coords):
    Bv, Lv = coords.shape[:2]
    Xf = coords.reshape(Bv, Lv * 3, 3)
    dX = Xf[:, 1:] - Xf[:, :-1]
    U = _unit(dX)
    u2, u1, u0 = U[:, :-2], U[:, 1:-1], U[:, 2:]
    n2 = _unit(jnp.cross(u2, u1))
    n1 = _unit(jnp.cross(u1, u0))
    cosD = jnp.clip(jnp.sum(n2 * n1, -1), -1 + 1e-7, 1 - 1e-7)
    D = jnp.sign(jnp.sum(u2 * n1, -1)) * jnp.arccos(cosD)
    D = jnp.pad(D, ((0, 0), (1, 2)))
    D = D.reshape(Bv, Lv, 3)
    return jnp.concatenate([jnp.cos(D), jnp.sin(D)], -1)


def _gvp_w(p):
    return dict(p, bs2=p["bs"].reshape(1, -1), bg2=p["bg"].reshape(1, -1))


def _bd3(A, pad_to=None):
    r, c = A.shape
    cp = c if pad_to is None else pad_to
    Z = jnp.zeros((3 * r, 3 * cp), F32)
    for k in range(3):
        Z = Z.at[k * r:(k + 1) * r, k * cp:k * cp + c].set(A)
    return Z


def _pack_small(q):
    Ws, Wg = q["Ws"], q["Wg"]
    Wse = jnp.concatenate([Ws, Ws @ Wg], axis=1)
    bse = jnp.concatenate([q["bs2"], q["bs2"] @ Wg + q["bg2"]], axis=1)
    return [_bd3(q["Wh"]), Wse, bse, _bd3(q["Wv"])]


def _pack_msg0(q):
    Wh, Ws, Wv, Wg = q["Wh"], q["Ws"], q["Wv"], q["Wg"]

    def sg(A):
        return jnp.concatenate([A, A @ Wg], axis=1)

    Wn0 = jnp.zeros((176, 243), F32)
    Wn0 = Wn0.at[0:128, 0:144].set(sg(Ws[0:128, :]))
    Wn0 = Wn0.at[128:176, 144:243].set(_bd3(Wh[0:16, :]))
    bn0 = jnp.zeros((1, 243), F32)
    bn0 = bn0.at[0, 0:144].set(
        jnp.concatenate([q["bs2"], q["bs2"] @ Wg + q["bg2"]], axis=1)[0])
    We0 = jnp.zeros((224, 243), F32)
    We0 = We0.at[0:128, 0:144].set(sg(Ws[128:256, :]))
    We0 = We0.at[128:176, 144:243].set(_bd3(Wh[16:32, :]))
    We0 = We0.at[176:208, 0:144].set(sg(Ws[256:288, :]))
    for c in range(3):
        We0 = We0.at[208 + c, 144 + 33 * c:177 + 33 * c].set(Wh[32, :])
    Wvn0 = sg(Ws[288:321, :])
    Wv0p = jnp.zeros((99, 48), F32)
    for c in range(3):
        Wv0p = Wv0p.at[33 * c:33 * c + 33, 16 * c:16 * c + 16].set(Wv)
    return [Wn0, bn0, We0, Wvn0, Wv0p]


def kernel(coords, coord_mask, padding_mask, params):
    Bv, Lv = coords.shape[:2]
    N = Bv * Lv
    K = 16
    CA = coords[:, :, 1, :]
    caf_pad = jnp.concatenate(
        [CA, jnp.zeros((Bv, Lv, 13), F32)], axis=-1)
    cat_pad = caf_pad.transpose(0, 2, 1)

    src = _topk(caf_pad, cat_pad, Lv, K).reshape(N * K)

    dih = _dih_feats(coords).reshape(N, 6)
    dih8 = jnp.concatenate([dih, jnp.zeros((N, 2), F32)], axis=1)
    fwdv = jnp.pad(_unit(CA[:, 1:] - CA[:, :-1]), ((0, 0), (0, 1), (0, 0)))
    bwdv = jnp.pad(_unit(CA[:, :-1] - CA[:, 1:]), ((0, 0), (1, 0), (0, 0)))
    ori = jnp.stack([fwdv, bwdv], axis=-1).reshape(N, 6)
    ori8 = jnp.concatenate([ori, jnp.zeros((N, 2), F32)], axis=1)
    caf_flat = caf_pad.reshape(N, 16)
    X = _node_embed(dih8, ori8, caf_flat, _gvp_w(params["node_embed"]))

    centers = jnp.linspace(0.0, 20.0, 16).reshape(1, 16).astype(F32)
    ep = _gvp_w(params["edge_embed"])
    ef = None
    for li, lp in enumerate(params["layers"]):
        m0, m1, m2 = (_gvp_w(q) for q in lp["msg"])
        f0, f1 = (_gvp_w(q) for q in lp["ff"])
        wlist = _pack_msg0(m0) + _pack_small(m1) + _pack_small(m2)
        wlist += [lp["ln1"]["g"].reshape(1, -1), lp["ln1"]["b"].reshape(1, -1)]
        wlist += _pack_small(f0) + _pack_small(f1)
        wlist += [lp["ln2"]["g"].reshape(1, -1), lp["ln2"]["b"].reshape(1, -1)]
        Xg = _sc_gather_rows(X, src.reshape(-1, 128))
        if li == 0:
            extra = (caf_flat, centers, ep["Wh"], ep["Ws"], ep["bs2"],
                     ep["Wv"], ep["Wg"].T, ep["bg2"])
        else:
            extra = (ef,)
        X, ef_new = _layer(X, Xg, wlist, extra)
        if li == 0:
            ef = ef_new

    s = X[:, 0:128].reshape(Bv, Lv, 128)
    V = jnp.stack([X[:, 128:144], X[:, 144:160], X[:, 160:176]],
                  axis=-1).reshape(Bv, Lv, 16, 3)
    return s, V


# speedup vs baseline: 1.0385x; 1.0385x over previous
import functools

import jax
import jax.numpy as jnp
from jax import lax
from jax.experimental import pallas as pl
from jax.experimental.pallas import tpu as pltpu
from jax.experimental.pallas import tpu_sc as plsc

NSC, NSUB = 2, 16
NWORK = NSC * NSUB
F32 = jnp.float32
XW = 256


def _sc_gather_rows(table, idx2):
    R, D = table.shape
    G, C = idx2.shape
    E = G * C
    iters = G // NWORK
    mesh = plsc.VectorSubcoreMesh(core_axis_name="c", subcore_axis_name="s")

    @functools.partial(
        pl.kernel,
        out_type=jax.ShapeDtypeStruct((E, D), F32),
        mesh=mesh,
        scratch_types=[
            pltpu.VMEM((iters, C), jnp.int32),
            pltpu.VMEM((C, D), F32),
            pltpu.VMEM((C, D), F32),
            pltpu.SemaphoreType.DMA,
            pltpu.SemaphoreType.DMA,
            pltpu.SemaphoreType.DMA,
            pltpu.SemaphoreType.DMA,
        ],
    )
    def gk(table_hbm, idx_hbm, out_hbm, idx_v, r0, r1, sg0, sg1, ss0, ss1):
        wid = lax.axis_index("s") * NSC + lax.axis_index("c")
        g0 = wid * iters
        pltpu.sync_copy(idx_hbm.at[pl.ds(g0, iters)], idx_v)
        rows = (r0, r1)
        semg = (sg0, sg1)
        sems = (ss0, ss1)
        hg = [None, None]
        hs = [None, None]
        for i in range(iters):
            b = i & 1
            if i >= 2:
                hs[b].wait()
            hg[b] = pltpu.async_copy(table_hbm.at[idx_v.at[i]], rows[b],
                                     semg[b])
            if i >= 1:
                p = 1 - b
                hg[p].wait()
                hs[p] = pltpu.async_copy(
                    rows[p], out_hbm.at[pl.ds((g0 + i - 1) * C, C)], sems[p])
        last = iters - 1
        hg[last & 1].wait()
        hs[last & 1] = pltpu.async_copy(
            rows[last & 1], out_hbm.at[pl.ds((g0 + last) * C, C)],
            sems[last & 1])
        if iters >= 2:
            hs[(last - 1) & 1].wait()
        hs[last & 1].wait()

    return gk(table, idx2)


def _topk_body(caf_ref, cat_ref, src_ref, *, L, RB, K):
    b = pl.program_id(0)
    r = pl.program_id(1)
    a = caf_ref[0]
    bt = cat_ref[0]
    d = jnp.zeros((RB, L), F32)
    for c in range(3):
        dc = a[:, c:c + 1] - bt[c:c + 1, :]
        d = d + dc * dc
    jcol = lax.broadcasted_iota(jnp.int32, (RB, L), 1).astype(F32)
    irow = (r * RB + lax.broadcasted_iota(jnp.int32, (RB, L), 0)).astype(F32)
    d = jnp.where(jcol == irow, 1e18, d)
    off = b * L
    for k in range(K):
        m = jnp.min(d, axis=1, keepdims=True)
        eq = d == m
        idxf = jnp.min(jnp.where(eq, jcol, 3e4), axis=1, keepdims=True)
        src_ref[0, :, k:k + 1] = idxf.astype(jnp.int32) + off
        d = jnp.where(eq, 1e17, d)


def _topk(caf_pad, cat_pad, L, K):
    B = caf_pad.shape[0]
    RB = 256
    return pl.pallas_call(
        functools.partial(_topk_body, L=L, RB=RB, K=K),
        grid=(B, L // RB),
        in_specs=[
            pl.BlockSpec((1, RB, 16), lambda b, r: (b, r, 0)),
            pl.BlockSpec((1, 16, L), lambda b, r: (b, 0, 0)),
        ],
        out_specs=pl.BlockSpec((1, RB, K), lambda b, r: (b, r, 0)),
        out_shape=jax.ShapeDtypeStruct((B, L, K), jnp.int32),
    )(caf_pad, cat_pad)


def _node_embed_body(dih_ref, ori_ref, caf_ref, Wh, Ws, bs, Wv, Wg, bg,
                     x_ref, *, NB):
    dih = dih_ref[...]
    ori = ori_ref[...]
    Vh = [jnp.dot(ori[:, 2 * c:2 * c + 2], Wh[...],
                  preferred_element_type=F32) for c in range(3)]
    vn = jnp.sqrt(Vh[0] * Vh[0] + Vh[1] * Vh[1] + Vh[2] * Vh[2] + 1e-8)
    sm = (jnp.dot(dih[:, 0:6], Ws[0:6, :], preferred_element_type=F32)
          + jnp.dot(vn, Ws[6:22, :], preferred_element_type=F32) + bs[...])
    gate = jax.nn.sigmoid(jnp.dot(sm, Wg[...], preferred_element_type=F32)
                          + bg[...])
    Vo = [jnp.dot(h, Wv[...], preferred_element_type=F32) * gate for h in Vh]
    pad = jnp.zeros((NB, XW - 192), F32)
    x_ref[...] = jnp.concatenate([sm] + Vo + [caf_ref[...], pad], axis=1)


def _node_embed(dih8, ori8, caf_flat, p):
    N = dih8.shape[0]
    NB = 256
    ws = (p["Wh"], p["Ws"], p["bs2"], p["Wv"], p["Wg"], p["bg2"])
    wspecs = [pl.BlockSpec(w.shape, lambda i: (0,) * w.ndim) for w in ws]
    return pl.pallas_call(
        functools.partial(_node_embed_body, NB=NB),
        grid=(N // NB,),
        in_specs=[pl.BlockSpec((NB, 8), lambda i: (i, 0)),
                  pl.BlockSpec((NB, 8), lambda i: (i, 0)),
                  pl.BlockSpec((NB, 16), lambda i: (i, 0))] + wspecs,
        out_specs=pl.BlockSpec((NB, XW), lambda i: (i, 0)),
        out_shape=jax.ShapeDtypeStruct((N, XW), F32),
    )(dih8, ori8, caf_flat, *ws)


def _gvp_packed(s, V48, Whb, Wse, bse, Wvb, act):
    Vh = jnp.dot(V48, Whb[...], preferred_element_type=F32)
    vn = jnp.sqrt(Vh[:, 0:16] * Vh[:, 0:16] + Vh[:, 16:32] * Vh[:, 16:32]
                  + Vh[:, 32:48] * Vh[:, 32:48] + 1e-8)
    smq = jnp.dot(jnp.concatenate([s, vn], axis=1), Wse[...],
                  preferred_element_type=F32) + bse[...]
    gate = jax.nn.sigmoid(smq[:, 128:144])
    g3 = jnp.concatenate([gate, gate, gate], axis=1)
    Vo = jnp.dot(Vh, Wvb[...], preferred_element_type=F32) * g3
    so = jax.nn.relu(smq[:, 0:128]) if act else smq[:, 0:128]
    return so, Vo


def _ln(s, g, b):
    m = jnp.mean(s, axis=1, keepdims=True)
    v = jnp.mean((s - m) * (s - m), axis=1, keepdims=True)
    return (s - m) / jnp.sqrt(v + 1e-5) * g[...] + b[...]


def _vln48(V48):
    n = jnp.sqrt(jnp.sum(V48 * V48, axis=1, keepdims=True) / 16.0 + 1e-4)
    return V48 / n


def _edge_gvp(Xg, caf_blk, cen, eWh, eWs, ebs, eWv, eWgT, ebg, *, NB):
    ne = NB * 16
    ca_d = jnp.broadcast_to(caf_blk[:, None, :], (NB, 16, 16)).reshape(ne, 16)
    dv = Xg[:, 176:192] - ca_d
    dist = jnp.sqrt(jnp.sum(dv * dv, axis=1, keepdims=True) + 1e-8)
    rbf = jnp.exp(-(((dist - cen[...]) / 1.25) ** 2))
    Vh = (dv / dist) * eWh[0, 0]
    vn = jnp.sqrt(jnp.sum(Vh * Vh, axis=1, keepdims=True) + 1e-8)
    sm = (jnp.dot(rbf, eWs[0:16, :], preferred_element_type=F32)
          + vn * eWs[16:17, :] + ebs[...])
    gate = jax.nn.sigmoid(
        jnp.sum(sm * eWgT[...], axis=1, keepdims=True) + ebg[...])
    eV16 = Vh * (eWv[0, 0] * gate)
    return sm, eV16


def _layer_body(first, NB, x_ref, xg_ref, *refs):
    if first:
        (caf_b, cen, eWh, eWs, ebs, eWv, eWg, ebg) = refs[:8]
        refs = refs[8:]
    else:
        ef_ref = refs[0]
        refs = refs[1:]
    (Wn0a, Wn0v, bn0, We0a, We0v, Wvn0, Wv0p,
     Whb1, Ws1, bs1, Wvb1,
     Whb2, Ws2, bs2, Wvb2,
     g1, b1,
     Whf0, Wsf0, bsf0, Wvf0,
     Whf1, Wsf1, bsf1, Wvf1,
     g2, b2) = refs[:27]
    outs = refs[27:]
    x_out = outs[0]
    ne = NB * 16
    X = x_ref[...]
    Xg = xg_ref[...]
    if first:
        es, eV16 = _edge_gvp(Xg, caf_b[...], cen, eWh, eWs, ebs, eWv, eWg,
                             ebg, NB=NB)
        efv = jnp.concatenate([es, eV16], axis=1)
        outs[1][...] = efv
    else:
        efv = ef_ref[...]
    s_d = X[:, 0:128]
    Vd48 = X[:, 128:176]

    def rep(t):
        return jnp.broadcast_to(t[:, None, :], (NB, 16, t.shape[1])
                                ).reshape(ne, t.shape[1])

    Mna = (jnp.dot(X[:, 0:176], Wn0a[...], preferred_element_type=F32)
           + bn0[...])
    Mnv = jnp.dot(X[:, 128:176], Wn0v[...], preferred_element_type=F32)
    ein = jnp.concatenate([Xg[:, 0:176], efv], axis=1)
    Vh = jnp.dot(ein, We0v[...], preferred_element_type=F32) + rep(Mnv)
    vn = jnp.sqrt(Vh[:, 0:40] * Vh[:, 0:40] + Vh[:, 40:80] * Vh[:, 40:80]
                  + Vh[:, 80:120] * Vh[:, 80:120] + 1e-8)
    smq = (jnp.dot(ein, We0a[...], preferred_element_type=F32) + rep(Mna)
           + jnp.dot(vn, Wvn0[...], preferred_element_type=F32))
    gate = jax.nn.sigmoid(smq[:, 128:144])
    ms = jax.nn.relu(smq[:, 0:128])
    mV = (jnp.dot(Vh, Wv0p[...], preferred_element_type=F32)
          * jnp.concatenate([gate, gate, gate], axis=1))
    ms, mV = _gvp_packed(ms, mV, Whb1, Ws1, bs1, Wvb1, True)
    ms, mV = _gvp_packed(ms, mV, Whb2, Ws2, bs2, Wvb2, False)
    ags = jnp.sum(ms.reshape(NB, 16, 128), axis=1) * (1.0 / 16.0)
    agV = jnp.sum(mV.reshape(NB, 16, 48), axis=1) * (1.0 / 16.0)
    s_n = _ln(s_d + ags, g1, b1)
    V_n = _vln48(Vd48 + agV)
    fs, fV = _gvp_packed(s_n, V_n, Whf0, Wsf0, bsf0, Wvf0, True)
    fs, fV = _gvp_packed(fs, fV, Whf1, Wsf1, bsf1, Wvf1, False)
    s_o = _ln(s_n + fs, g2, b2)
    V_o = _vln48(V_n + fV)
    x_out[...] = jnp.concatenate([s_o, V_o, X[:, 176:XW]], axis=1)


def _layer(X, Xg, wlist, extra):
    N = X.shape[0]
    E = Xg.shape[0]
    NB = 256
    first = len(extra) > 1
    if first:
        especs = ([pl.BlockSpec((NB, 16), lambda i: (i, 0)),
                   pl.BlockSpec((1, 16), lambda i: (0, 0))]
                  + [pl.BlockSpec(w.shape, lambda i: (0,) * w.ndim)
                     for w in extra[2:]])
        out_specs = [pl.BlockSpec((NB, XW), lambda i: (i, 0)),
                     pl.BlockSpec((NB * 16, 48), lambda i: (i, 0))]
        out_shape = [jax.ShapeDtypeStruct((N, XW), F32),
                     jax.ShapeDtypeStruct((E, 48), F32)]
    else:
        especs = [pl.BlockSpec((NB * 16, 48), lambda i: (i, 0))]
        out_specs = [pl.BlockSpec((NB, XW), lambda i: (i, 0))]
        out_shape = [jax.ShapeDtypeStruct((N, XW), F32)]
    wspecs = [pl.BlockSpec(w.shape, lambda i: (0,) * w.ndim) for w in wlist]
    res = pl.pallas_call(
        functools.partial(_layer_body, first, NB),
        grid=(N // NB,),
        in_specs=[pl.BlockSpec((NB, XW), lambda i: (i, 0)),
                  pl.BlockSpec((NB * 16, XW), lambda i: (i, 0))]
        + especs + wspecs,
        out_specs=out_specs,
        out_shape=out_shape,
    )(X, Xg, *extra, *wlist)
    return res if first else (res[0], None)


def _unit(v, axis=-1, eps=1e-8):
    return v / jnp.sqrt(jnp.sum(v * v, axis=axis, keepdims=True) + eps)


def _dih_feats(coords):
    Bv, Lv = coords.shape[:2]
    Xf = coords.reshape(Bv, Lv * 3, 3)
    dX = Xf[:, 1:] - Xf[:, :-1]
    U = _unit(dX)
    u2, u1, u0 = U[:, :-2], U[:, 1:-1], U[:, 2:]
    n2 = _unit(jnp.cross(u2, u1))
    n1 = _unit(jnp.cross(u1, u0))
    cosD = jnp.clip(jnp.sum(n2 * n1, -1), -1 + 1e-7, 1 - 1e-7)
    D = jnp.sign(jnp.sum(u2 * n1, -1)) * jnp.arccos(cosD)
    D = jnp.pad(D, ((0, 0), (1, 2)))
    D = D.reshape(Bv, Lv, 3)
    return jnp.concatenate([jnp.cos(D), jnp.sin(D)], -1)


def _gvp_w(p):
    return dict(p, bs2=p["bs"].reshape(1, -1), bg2=p["bg"].reshape(1, -1))


def _bd3(A, pad_to=None):
    r, c = A.shape
    cp = c if pad_to is None else pad_to
    Z = jnp.zeros((3 * r, 3 * cp), F32)
    for k in range(3):
        Z = Z.at[k * r:(k + 1) * r, k * cp:k * cp + c].set(A)
    return Z


def _pack_small(q):
    Ws, Wg = q["Ws"], q["Wg"]
    Wse = jnp.concatenate([Ws, Ws @ Wg], axis=1)
    bse = jnp.concatenate([q["bs2"], q["bs2"] @ Wg + q["bg2"]], axis=1)
    return [_bd3(q["Wh"]), Wse, bse, _bd3(q["Wv"])]


def _pack_msg0(q):
    Wh, Ws, Wv, Wg = q["Wh"], q["Ws"], q["Wv"], q["Wg"]

    def sg(A):
        return jnp.concatenate([A, A @ Wg], axis=1)

    Wn0a = jnp.zeros((176, 144), F32).at[0:128, :].set(sg(Ws[0:128, :]))
    Wn0v = _bd3(Wh[0:16, :], pad_to=40)
    bn0 = jnp.concatenate([q["bs2"], q["bs2"] @ Wg + q["bg2"]], axis=1)
    We0a = jnp.zeros((224, 144), F32)
    We0a = We0a.at[0:128, :].set(sg(Ws[128:256, :]))
    We0a = We0a.at[176:208, :].set(sg(Ws[256:288, :]))
    We0v = jnp.zeros((224, 120), F32)
    We0v = We0v.at[128:176, :].set(_bd3(Wh[16:32, :], pad_to=40))
    for c in range(3):
        We0v = We0v.at[208 + c, 40 * c:40 * c + 33].set(Wh[32, :])
    Wvn0 = jnp.zeros((40, 144), F32).at[0:33, :].set(sg(Ws[288:321, :]))
    Wv0p = jnp.zeros((120, 48), F32)
    for c in range(3):
        Wv0p = Wv0p.at[40 * c:40 * c + 33, 16 * c:16 * c + 16].set(Wv)
    return [Wn0a, Wn0v, bn0, We0a, We0v, Wvn0, Wv0p]


def kernel(coords, coord_mask, padding_mask, params):
    Bv, Lv = coords.shape[:2]
    N = Bv * Lv
    K = 16
    CA = coords[:, :, 1, :]
    caf_pad = jnp.concatenate(
        [CA, jnp.zeros((Bv, Lv, 13), F32)], axis=-1)
    cat_pad = caf_pad.transpose(0, 2, 1)

    src = _topk(caf_pad, cat_pad, Lv, K).reshape(N * K)

    dih = _dih_feats(coords).reshape(N, 6)
    dih8 = jnp.concatenate([dih, jnp.zeros((N, 2), F32)], axis=1)
    fwdv = jnp.pad(_unit(CA[:, 1:] - CA[:, :-1]), ((0, 0), (0, 1), (0, 0)))
    bwdv = jnp.pad(_unit(CA[:, :-1] - CA[:, 1:]), ((0, 0), (1, 0), (0, 0)))
    ori = jnp.stack([fwdv, bwdv], axis=-1).reshape(N, 6)
    ori8 = jnp.concatenate([ori, jnp.zeros((N, 2), F32)], axis=1)
    caf_flat = caf_pad.reshape(N, 16)
    X = _node_embed(dih8, ori8, caf_flat, _gvp_w(params["node_embed"]))

    centers = jnp.linspace(0.0, 20.0, 16).reshape(1, 16).astype(F32)
    ep = _gvp_w(params["edge_embed"])
    ef = None
    for li, lp in enumerate(params["layers"]):
        m0, m1, m2 = (_gvp_w(q) for q in lp["msg"])
        f0, f1 = (_gvp_w(q) for q in lp["ff"])
        wlist = _pack_msg0(m0) + _pack_small(m1) + _pack_small(m2)
        wlist += [lp["ln1"]["g"].reshape(1, -1), lp["ln1"]["b"].reshape(1, -1)]
        wlist += _pack_small(f0) + _pack_small(f1)
        wlist += [lp["ln2"]["g"].reshape(1, -1), lp["ln2"]["b"].reshape(1, -1)]
        Xg = _sc_gather_rows(X, src.reshape(-1, 128))
        if li == 0:
            extra = (caf_flat, centers, ep["Wh"], ep["Ws"], ep["bs2"],
                     ep["Wv"], ep["Wg"].T, ep["bg2"])
        else:
            extra = (ef,)
        X, ef_new = _layer(X, Xg, wlist, extra)
        if li == 0:
            ef = ef_new

    s = X[:, 0:128].reshape(Bv, Lv, 128)
    V = jnp.stack([X[:, 128:144], X[:, 144:160], X[:, 160:176]],
                  axis=-1).reshape(Bv, Lv, 16, 3)
    return s, V


# speedup vs baseline: 1.0470x; 1.0082x over previous
import functools

import jax
import jax.numpy as jnp
from jax import lax
from jax.experimental import pallas as pl
from jax.experimental.pallas import tpu as pltpu
from jax.experimental.pallas import tpu_sc as plsc

NSC, NSUB = 2, 16
NWORK = NSC * NSUB
F32 = jnp.float32
XW = 256


def _sc_gather_rows(table, idx2):
    R, D = table.shape
    G, C = idx2.shape
    E = G * C
    iters = G // NWORK
    mesh = plsc.VectorSubcoreMesh(core_axis_name="c", subcore_axis_name="s")

    @functools.partial(
        pl.kernel,
        out_type=jax.ShapeDtypeStruct((E, D), F32),
        mesh=mesh,
        scratch_types=[
            pltpu.VMEM((iters, C), jnp.int32),
            pltpu.VMEM((C, D), F32),
            pltpu.VMEM((C, D), F32),
            pltpu.SemaphoreType.DMA,
            pltpu.SemaphoreType.DMA,
            pltpu.SemaphoreType.DMA,
            pltpu.SemaphoreType.DMA,
        ],
    )
    def gk(table_hbm, idx_hbm, out_hbm, idx_v, r0, r1, sg0, sg1, ss0, ss1):
        wid = lax.axis_index("s") * NSC + lax.axis_index("c")
        g0 = wid * iters
        pltpu.sync_copy(idx_hbm.at[pl.ds(g0, iters)], idx_v)
        rows = (r0, r1)
        semg = (sg0, sg1)
        sems = (ss0, ss1)
        hg = [None, None]
        hs = [None, None]
        for i in range(iters):
            b = i & 1
            if i >= 2:
                hs[b].wait()
            hg[b] = pltpu.async_copy(table_hbm.at[idx_v.at[i]], rows[b],
                                     semg[b])
            if i >= 1:
                p = 1 - b
                hg[p].wait()
                hs[p] = pltpu.async_copy(
                    rows[p], out_hbm.at[pl.ds((g0 + i - 1) * C, C)], sems[p])
        last = iters - 1
        hg[last & 1].wait()
        hs[last & 1] = pltpu.async_copy(
            rows[last & 1], out_hbm.at[pl.ds((g0 + last) * C, C)],
            sems[last & 1])
        if iters >= 2:
            hs[(last - 1) & 1].wait()
        hs[last & 1].wait()

    return gk(table, idx2)


def _topk_body(caf_ref, cat_ref, src_ref, *, L, RB, K):
    b = pl.program_id(0)
    r = pl.program_id(1)
    a = caf_ref[0]
    bt = cat_ref[0]
    d = jnp.zeros((RB, L), F32)
    for c in range(3):
        dc = a[:, c:c + 1] - bt[c:c + 1, :]
        d = d + dc * dc
    jcol = lax.broadcasted_iota(jnp.int32, (RB, L), 1).astype(F32)
    irow = (r * RB + lax.broadcasted_iota(jnp.int32, (RB, L), 0)).astype(F32)
    d = jnp.where(jcol == irow, 1e18, d)
    off = b * L
    for k in range(K):
        m = jnp.min(d, axis=1, keepdims=True)
        eq = d == m
        idxf = jnp.min(jnp.where(eq, jcol, 3e4), axis=1, keepdims=True)
        src_ref[0, :, k:k + 1] = idxf.astype(jnp.int32) + off
        d = jnp.where(eq, 1e17, d)


def _topk(caf_pad, cat_pad, L, K):
    B = caf_pad.shape[0]
    RB = 256
    return pl.pallas_call(
        functools.partial(_topk_body, L=L, RB=RB, K=K),
        grid=(B, L // RB),
        in_specs=[
            pl.BlockSpec((1, RB, 16), lambda b, r: (b, r, 0)),
            pl.BlockSpec((1, 16, L), lambda b, r: (b, 0, 0)),
        ],
        out_specs=pl.BlockSpec((1, RB, K), lambda b, r: (b, r, 0)),
        out_shape=jax.ShapeDtypeStruct((B, L, K), jnp.int32),
    )(caf_pad, cat_pad)


def _node_embed_body(dih_ref, ori_ref, caf_ref, Wh, Ws, bs, Wv, Wg, bg,
                     x_ref, *, NB):
    dih = dih_ref[...]
    ori = ori_ref[...]
    Vh = [jnp.dot(ori[:, 2 * c:2 * c + 2], Wh[...],
                  preferred_element_type=F32) for c in range(3)]
    vn = jnp.sqrt(Vh[0] * Vh[0] + Vh[1] * Vh[1] + Vh[2] * Vh[2] + 1e-8)
    sm = (jnp.dot(dih[:, 0:6], Ws[0:6, :], preferred_element_type=F32)
          + jnp.dot(vn, Ws[6:22, :], preferred_element_type=F32) + bs[...])
    gate = jax.nn.sigmoid(jnp.dot(sm, Wg[...], preferred_element_type=F32)
                          + bg[...])
    Vo = [jnp.dot(h, Wv[...], preferred_element_type=F32) * gate for h in Vh]
    pad = jnp.zeros((NB, XW - 192), F32)
    x_ref[...] = jnp.concatenate([sm] + Vo + [caf_ref[...], pad], axis=1)


def _node_embed(dih8, ori8, caf_flat, p):
    N = dih8.shape[0]
    NB = 256
    ws = (p["Wh"], p["Ws"], p["bs2"], p["Wv"], p["Wg"], p["bg2"])
    wspecs = [pl.BlockSpec(w.shape, lambda i: (0,) * w.ndim) for w in ws]
    return pl.pallas_call(
        functools.partial(_node_embed_body, NB=NB),
        grid=(N // NB,),
        in_specs=[pl.BlockSpec((NB, 8), lambda i: (i, 0)),
                  pl.BlockSpec((NB, 8), lambda i: (i, 0)),
                  pl.BlockSpec((NB, 16), lambda i: (i, 0))] + wspecs,
        out_specs=pl.BlockSpec((NB, XW), lambda i: (i, 0)),
        out_shape=jax.ShapeDtypeStruct((N, XW), F32),
    )(dih8, ori8, caf_flat, *ws)


def _gvp_packed(s, V48, Whb, Wse, bse, Wvb, act):
    Vh = jnp.dot(V48, Whb[...], preferred_element_type=F32)
    vn = jnp.sqrt(Vh[:, 0:16] * Vh[:, 0:16] + Vh[:, 16:32] * Vh[:, 16:32]
                  + Vh[:, 32:48] * Vh[:, 32:48] + 1e-8)
    smq = jnp.dot(jnp.concatenate([s, vn], axis=1), Wse[...],
                  preferred_element_type=F32) + bse[...]
    gate = jax.nn.sigmoid(smq[:, 128:144])
    g3 = jnp.concatenate([gate, gate, gate], axis=1)
    Vo = jnp.dot(Vh, Wvb[...], preferred_element_type=F32) * g3
    so = jax.nn.relu(smq[:, 0:128]) if act else smq[:, 0:128]
    return so, Vo


def _ln(s, g, b):
    m = jnp.mean(s, axis=1, keepdims=True)
    v = jnp.mean((s - m) * (s - m), axis=1, keepdims=True)
    return (s - m) / jnp.sqrt(v + 1e-5) * g[...] + b[...]


def _vln48(V48):
    n = jnp.sqrt(jnp.sum(V48 * V48, axis=1, keepdims=True) / 16.0 + 1e-4)
    return V48 / n


def _edge_gvp(Xg, caf_blk, cen, eWh, eWs, ebs, eWv, eWgT, ebg, *, NB):
    ne = NB * 16
    ca_d = jnp.broadcast_to(caf_blk[:, None, :], (NB, 16, 16)).reshape(ne, 16)
    dv = Xg[:, 176:192] - ca_d
    dist = jnp.sqrt(jnp.sum(dv * dv, axis=1, keepdims=True) + 1e-8)
    rbf = jnp.exp(-(((dist - cen[...]) / 1.25) ** 2))
    Vh = (dv / dist) * eWh[0, 0]
    vn = jnp.sqrt(jnp.sum(Vh * Vh, axis=1, keepdims=True) + 1e-8)
    sm = (jnp.dot(rbf, eWs[0:16, :], preferred_element_type=F32)
          + vn * eWs[16:17, :] + ebs[...])
    gate = jax.nn.sigmoid(
        jnp.sum(sm * eWgT[...], axis=1, keepdims=True) + ebg[...])
    eV16 = Vh * (eWv[0, 0] * gate)
    return sm, eV16


def _layer_body(first, NB, x_ref, xg_ref, *refs):
    if first:
        (caf_b, cen, eWh, eWs, ebs, eWv, eWg, ebg) = refs[:8]
        refs = refs[8:]
    else:
        ef_ref = refs[0]
        refs = refs[1:]
    (Wn0, bn0, We0, Wvn0, Wg0, bg0, Wv0p,
     Whb1, Ws1, bs1, Wvb1,
     Whb2, Ws2, bs2, Wvb2,
     g1, b1,
     Whf0, Wsf0, bsf0, Wvf0,
     Whf1, Wsf1, bsf1, Wvf1,
     g2, b2) = refs[:27]
    outs = refs[27:]
    x_out = outs[0]
    ne = NB * 16
    X = x_ref[...]
    Xg = xg_ref[...]
    if first:
        es, eV16 = _edge_gvp(Xg, caf_b[...], cen, eWh, eWs, ebs, eWv, eWg,
                             ebg, NB=NB)
        efv = jnp.concatenate([es, eV16], axis=1)
        outs[1][...] = efv
    else:
        efv = ef_ref[...]
    s_d = X[:, 0:128]
    Vd48 = X[:, 128:176]

    def rep(t):
        return jnp.broadcast_to(t[:, None, :], (NB, 16, t.shape[1])
                                ).reshape(ne, t.shape[1])

    Mn = jnp.dot(X[:, 0:176], Wn0[...], preferred_element_type=F32) + bn0[...]
    ein = jnp.concatenate([Xg[:, 0:176], efv], axis=1)
    M = jnp.dot(ein, We0[...], preferred_element_type=F32) + rep(Mn)
    Vh = M[:, 128:248]
    vn = jnp.sqrt(Vh[:, 0:40] * Vh[:, 0:40] + Vh[:, 40:80] * Vh[:, 40:80]
                  + Vh[:, 80:120] * Vh[:, 80:120] + 1e-8)
    sm = M[:, 0:128] + jnp.dot(vn, Wvn0[...], preferred_element_type=F32)
    gate = jax.nn.sigmoid(jnp.dot(sm, Wg0[...], preferred_element_type=F32)
                          + bg0[...])
    ms = jax.nn.relu(sm)
    mV = (jnp.dot(Vh, Wv0p[...], preferred_element_type=F32)
          * jnp.concatenate([gate, gate, gate], axis=1))
    ms, mV = _gvp_packed(ms, mV, Whb1, Ws1, bs1, Wvb1, True)
    ms, mV = _gvp_packed(ms, mV, Whb2, Ws2, bs2, Wvb2, False)
    ags = jnp.sum(ms.reshape(NB, 16, 128), axis=1) * (1.0 / 16.0)
    agV = jnp.sum(mV.reshape(NB, 16, 48), axis=1) * (1.0 / 16.0)
    s_n = _ln(s_d + ags, g1, b1)
    V_n = _vln48(Vd48 + agV)
    fs, fV = _gvp_packed(s_n, V_n, Whf0, Wsf0, bsf0, Wvf0, True)
    fs, fV = _gvp_packed(fs, fV, Whf1, Wsf1, bsf1, Wvf1, False)
    s_o = _ln(s_n + fs, g2, b2)
    V_o = _vln48(V_n + fV)
    x_out[...] = jnp.concatenate([s_o, V_o, X[:, 176:XW]], axis=1)


def _layer(X, Xg, wlist, extra):
    N = X.shape[0]
    E = Xg.shape[0]
    NB = 256
    first = len(extra) > 1
    if first:
        especs = ([pl.BlockSpec((NB, 16), lambda i: (i, 0)),
                   pl.BlockSpec((1, 16), lambda i: (0, 0))]
                  + [pl.BlockSpec(w.shape, lambda i: (0,) * w.ndim)
                     for w in extra[2:]])
        out_specs = [pl.BlockSpec((NB, XW), lambda i: (i, 0)),
                     pl.BlockSpec((NB * 16, 48), lambda i: (i, 0))]
        out_shape = [jax.ShapeDtypeStruct((N, XW), F32),
                     jax.ShapeDtypeStruct((E, 48), F32)]
    else:
        especs = [pl.BlockSpec((NB * 16, 48), lambda i: (i, 0))]
        out_specs = [pl.BlockSpec((NB, XW), lambda i: (i, 0))]
        out_shape = [jax.ShapeDtypeStruct((N, XW), F32)]
    wspecs = [pl.BlockSpec(w.shape, lambda i: (0,) * w.ndim) for w in wlist]
    res = pl.pallas_call(
        functools.partial(_layer_body, first, NB),
        grid=(N // NB,),
        in_specs=[pl.BlockSpec((NB, XW), lambda i: (i, 0)),
                  pl.BlockSpec((NB * 16, XW), lambda i: (i, 0))]
        + especs + wspecs,
        out_specs=out_specs,
        out_shape=out_shape,
    )(X, Xg, *extra, *wlist)
    return res if first else (res[0], None)


def _unit(v, axis=-1, eps=1e-8):
    return v / jnp.sqrt(jnp.sum(v * v, axis=axis, keepdims=True) + eps)


def _dih_feats(coords):
    Bv, Lv = coords.shape[:2]
    Xf = coords.reshape(Bv, Lv * 3, 3)
    dX = Xf[:, 1:] - Xf[:, :-1]
    U = _unit(dX)
    u2, u1, u0 = U[:, :-2], U[:, 1:-1], U[:, 2:]
    n2 = _unit(jnp.cross(u2, u1))
    n1 = _unit(jnp.cross(u1, u0))
    cosD = jnp.clip(jnp.sum(n2 * n1, -1), -1 + 1e-7, 1 - 1e-7)
    D = jnp.sign(jnp.sum(u2 * n1, -1)) * jnp.arccos(cosD)
    D = jnp.pad(D, ((0, 0), (1, 2)))
    D = D.reshape(Bv, Lv, 3)
    return jnp.concatenate([jnp.cos(D), jnp.sin(D)], -1)


def _gvp_w(p):
    return dict(p, bs2=p["bs"].reshape(1, -1), bg2=p["bg"].reshape(1, -1))


def _bd3(A, pad_to=None):
    r, c = A.shape
    cp = c if pad_to is None else pad_to
    Z = jnp.zeros((3 * r, 3 * cp), F32)
    for k in range(3):
        Z = Z.at[k * r:(k + 1) * r, k * cp:k * cp + c].set(A)
    return Z


def _pack_small(q):
    Ws, Wg = q["Ws"], q["Wg"]
    Wse = jnp.concatenate([Ws, Ws @ Wg], axis=1)
    bse = jnp.concatenate([q["bs2"], q["bs2"] @ Wg + q["bg2"]], axis=1)
    return [_bd3(q["Wh"]), Wse, bse, _bd3(q["Wv"])]


def _pack_msg0(q):
    Wh, Ws, Wv, Wg = q["Wh"], q["Ws"], q["Wv"], q["Wg"]

    def sg(A):
        return jnp.concatenate([A, A @ Wg], axis=1)

    del sg
    Wn0 = jnp.zeros((176, 248), F32)
    Wn0 = Wn0.at[0:128, 0:128].set(Ws[0:128, :])
    Wn0 = Wn0.at[128:176, 128:248].set(_bd3(Wh[0:16, :], pad_to=40))
    bn0 = jnp.zeros((1, 248), F32).at[0, 0:128].set(q["bs"])
    We0 = jnp.zeros((224, 248), F32)
    We0 = We0.at[0:128, 0:128].set(Ws[128:256, :])
    We0 = We0.at[128:176, 128:248].set(_bd3(Wh[16:32, :], pad_to=40))
    We0 = We0.at[176:208, 0:128].set(Ws[256:288, :])
    for c in range(3):
        We0 = We0.at[208 + c, 128 + 40 * c:161 + 40 * c].set(Wh[32, :])
    Wvn0 = jnp.zeros((40, 128), F32).at[0:33, :].set(Ws[288:321, :])
    Wv0p = jnp.zeros((120, 48), F32)
    for c in range(3):
        Wv0p = Wv0p.at[40 * c:40 * c + 33, 16 * c:16 * c + 16].set(Wv)
    return [Wn0, bn0, We0, Wvn0, q["Wg"], q["bg2"], Wv0p]


def kernel(coords, coord_mask, padding_mask, params):
    Bv, Lv = coords.shape[:2]
    N = Bv * Lv
    K = 16
    CA = coords[:, :, 1, :]
    caf_pad = jnp.concatenate(
        [CA, jnp.zeros((Bv, Lv, 13), F32)], axis=-1)
    cat_pad = caf_pad.transpose(0, 2, 1)

    src = _topk(caf_pad, cat_pad, Lv, K).reshape(N * K)

    dih = _dih_feats(coords).reshape(N, 6)
    dih8 = jnp.concatenate([dih, jnp.zeros((N, 2), F32)], axis=1)
    fwdv = jnp.pad(_unit(CA[:, 1:] - CA[:, :-1]), ((0, 0), (0, 1), (0, 0)))
    bwdv = jnp.pad(_unit(CA[:, :-1] - CA[:, 1:]), ((0, 0), (1, 0), (0, 0)))
    ori = jnp.stack([fwdv, bwdv], axis=-1).reshape(N, 6)
    ori8 = jnp.concatenate([ori, jnp.zeros((N, 2), F32)], axis=1)
    caf_flat = caf_pad.reshape(N, 16)
    X = _node_embed(dih8, ori8, caf_flat, _gvp_w(params["node_embed"]))

    centers = jnp.linspace(0.0, 20.0, 16).reshape(1, 16).astype(F32)
    ep = _gvp_w(params["edge_embed"])
    ef = None
    for li, lp in enumerate(params["layers"]):
        m0, m1, m2 = (_gvp_w(q) for q in lp["msg"])
        f0, f1 = (_gvp_w(q) for q in lp["ff"])
        wlist = _pack_msg0(m0) + _pack_small(m1) + _pack_small(m2)
        wlist += [lp["ln1"]["g"].reshape(1, -1), lp["ln1"]["b"].reshape(1, -1)]
        wlist += _pack_small(f0) + _pack_small(f1)
        wlist += [lp["ln2"]["g"].reshape(1, -1), lp["ln2"]["b"].reshape(1, -1)]
        Xg = _sc_gather_rows(X, src.reshape(-1, 128))
        if li == 0:
            extra = (caf_flat, centers, ep["Wh"], ep["Ws"], ep["bs2"],
                     ep["Wv"], ep["Wg"].T, ep["bg2"])
        else:
            extra = (ef,)
        X, ef_new = _layer(X, Xg, wlist, extra)
        if li == 0:
            ef = ef_new

    s = X[:, 0:128].reshape(Bv, Lv, 128)
    V = jnp.stack([X[:, 128:144], X[:, 144:160], X[:, 160:176]],
                  axis=-1).reshape(Bv, Lv, 16, 3)
    return s, V


# speedup vs baseline: 1.0655x; 1.0177x over previous
import functools

import jax
import jax.numpy as jnp
from jax import lax
from jax.experimental import pallas as pl
from jax.experimental.pallas import tpu as pltpu
from jax.experimental.pallas import tpu_sc as plsc

NSC, NSUB = 2, 16
NWORK = NSC * NSUB
F32 = jnp.float32
XW = 256


def _sc_gather_rows(table, idx2):
    R, D = table.shape
    G, C = idx2.shape
    E = G * C
    iters = G // NWORK
    mesh = plsc.VectorSubcoreMesh(core_axis_name="c", subcore_axis_name="s")

    @functools.partial(
        pl.kernel,
        out_type=jax.ShapeDtypeStruct((E, D), F32),
        mesh=mesh,
        scratch_types=[
            pltpu.VMEM((iters, C), jnp.int32),
            pltpu.VMEM((C, D), F32),
            pltpu.VMEM((C, D), F32),
            pltpu.SemaphoreType.DMA,
            pltpu.SemaphoreType.DMA,
            pltpu.SemaphoreType.DMA,
            pltpu.SemaphoreType.DMA,
        ],
    )
    def gk(table_hbm, idx_hbm, out_hbm, idx_v, r0, r1, sg0, sg1, ss0, ss1):
        wid = lax.axis_index("s") * NSC + lax.axis_index("c")
        g0 = wid * iters
        pltpu.sync_copy(idx_hbm.at[pl.ds(g0, iters)], idx_v)
        rows = (r0, r1)
        semg = (sg0, sg1)
        sems = (ss0, ss1)
        hg = [None, None]
        hs = [None, None]
        for i in range(iters):
            b = i & 1
            if i >= 2:
                hs[b].wait()
            hg[b] = pltpu.async_copy(table_hbm.at[idx_v.at[i]], rows[b],
                                     semg[b])
            if i >= 1:
                p = 1 - b
                hg[p].wait()
                hs[p] = pltpu.async_copy(
                    rows[p], out_hbm.at[pl.ds((g0 + i - 1) * C, C)], sems[p])
        last = iters - 1
        hg[last & 1].wait()
        hs[last & 1] = pltpu.async_copy(
            rows[last & 1], out_hbm.at[pl.ds((g0 + last) * C, C)],
            sems[last & 1])
        if iters >= 2:
            hs[(last - 1) & 1].wait()
        hs[last & 1].wait()

    return gk(table, idx2)


def _topk_body(caf_ref, cat_ref, src_ref, *, L, RB, K):
    b = pl.program_id(0)
    r = pl.program_id(1)
    a = caf_ref[0]
    bt = cat_ref[0]
    d = jnp.zeros((RB, L), F32)
    for c in range(3):
        dc = a[:, c:c + 1] - bt[c:c + 1, :]
        d = d + dc * dc
    jcol = lax.broadcasted_iota(jnp.int32, (RB, L), 1).astype(F32)
    irow = (r * RB + lax.broadcasted_iota(jnp.int32, (RB, L), 0)).astype(F32)
    d = jnp.where(jcol == irow, 1e18, d)
    off = b * L
    for k in range(K):
        m = jnp.min(d, axis=1, keepdims=True)
        eq = d == m
        idxf = jnp.min(jnp.where(eq, jcol, 3e4), axis=1, keepdims=True)
        src_ref[0, :, k:k + 1] = idxf.astype(jnp.int32) + off
        d = jnp.where(eq, 1e17, d)


def _topk(caf_pad, cat_pad, L, K):
    B = caf_pad.shape[0]
    RB = 256
    return pl.pallas_call(
        functools.partial(_topk_body, L=L, RB=RB, K=K),
        grid=(B, L // RB),
        in_specs=[
            pl.BlockSpec((1, RB, 16), lambda b, r: (b, r, 0)),
            pl.BlockSpec((1, 16, L), lambda b, r: (b, 0, 0)),
        ],
        out_specs=pl.BlockSpec((1, RB, K), lambda b, r: (b, r, 0)),
        out_shape=jax.ShapeDtypeStruct((B, L, K), jnp.int32),
    )(caf_pad, cat_pad)


def _node_embed_body(dih_ref, ori_ref, caf_ref, Wh, Ws, bs, Wv, Wg, bg,
                     x_ref, *, NB):
    dih = dih_ref[...]
    ori = ori_ref[...]
    Vh = [jnp.dot(ori[:, 2 * c:2 * c + 2], Wh[...],
                  preferred_element_type=F32) for c in range(3)]
    vn = jnp.sqrt(Vh[0] * Vh[0] + Vh[1] * Vh[1] + Vh[2] * Vh[2] + 1e-8)
    sm = (jnp.dot(dih[:, 0:6], Ws[0:6, :], preferred_element_type=F32)
          + jnp.dot(vn, Ws[6:22, :], preferred_element_type=F32) + bs[...])
    gate = jax.nn.sigmoid(jnp.dot(sm, Wg[...], preferred_element_type=F32)
                          + bg[...])
    Vo = [jnp.dot(h, Wv[...], preferred_element_type=F32) * gate for h in Vh]
    pad = jnp.zeros((NB, XW - 192), F32)
    x_ref[...] = jnp.concatenate([sm] + Vo + [caf_ref[...], pad], axis=1)


def _node_embed(dih8, ori8, caf_flat, p):
    N = dih8.shape[0]
    NB = 256
    ws = (p["Wh"], p["Ws"], p["bs2"], p["Wv"], p["Wg"], p["bg2"])
    wspecs = [pl.BlockSpec(w.shape, lambda i: (0,) * w.ndim) for w in ws]
    return pl.pallas_call(
        functools.partial(_node_embed_body, NB=NB),
        grid=(N // NB,),
        in_specs=[pl.BlockSpec((NB, 8), lambda i: (i, 0)),
                  pl.BlockSpec((NB, 8), lambda i: (i, 0)),
                  pl.BlockSpec((NB, 16), lambda i: (i, 0))] + wspecs,
        out_specs=pl.BlockSpec((NB, XW), lambda i: (i, 0)),
        out_shape=jax.ShapeDtypeStruct((N, XW), F32),
    )(dih8, ori8, caf_flat, *ws)


def _gvp_packed(s, V48, Whb, Ws, bs, Wvb, Wg, bg, act):
    Vh = jnp.dot(V48, Whb[...], preferred_element_type=F32)
    vn = jnp.sqrt(Vh[:, 0:16] * Vh[:, 0:16] + Vh[:, 16:32] * Vh[:, 16:32]
                  + Vh[:, 32:48] * Vh[:, 32:48] + 1e-8)
    sm = jnp.dot(jnp.concatenate([s, vn], axis=1), Ws[...],
                 preferred_element_type=F32) + bs[...]
    gate = jax.nn.sigmoid(jnp.dot(sm, Wg[...], preferred_element_type=F32)
                          + bg[...])
    g3 = jnp.concatenate([gate, gate, gate], axis=1)
    Vo = jnp.dot(Vh, Wvb[...], preferred_element_type=F32) * g3
    so = jax.nn.relu(sm) if act else sm
    return so, Vo


def _ln(s, g, b):
    m = jnp.mean(s, axis=1, keepdims=True)
    v = jnp.mean((s - m) * (s - m), axis=1, keepdims=True)
    return (s - m) / jnp.sqrt(v + 1e-5) * g[...] + b[...]


def _vln48(V48):
    n = jnp.sqrt(jnp.sum(V48 * V48, axis=1, keepdims=True) / 16.0 + 1e-4)
    return V48 / n


def _edge_gvp(Xg, caf_blk, cen, eWh, eWs, ebs, eWv, eWgT, ebg, *, NB):
    ne = NB * 16
    ca_d = jnp.broadcast_to(caf_blk[:, None, :], (NB, 16, 16)).reshape(ne, 16)
    dv = Xg[:, 176:192] - ca_d
    dist = jnp.sqrt(jnp.sum(dv * dv, axis=1, keepdims=True) + 1e-8)
    rbf = jnp.exp(-(((dist - cen[...]) / 1.25) ** 2))
    Vh = (dv / dist) * eWh[0, 0]
    vn = jnp.sqrt(jnp.sum(Vh * Vh, axis=1, keepdims=True) + 1e-8)
    sm = (jnp.dot(rbf, eWs[0:16, :], preferred_element_type=F32)
          + vn * eWs[16:17, :] + ebs[...])
    gate = jax.nn.sigmoid(
        jnp.sum(sm * eWgT[...], axis=1, keepdims=True) + ebg[...])
    eV16 = Vh * (eWv[0, 0] * gate)
    return sm, eV16


def _layer_body(first, NB, x_ref, xg_ref, *refs):
    if first:
        (caf_b, cen, eWh, eWs, ebs, eWv, eWg, ebg) = refs[:8]
        refs = refs[8:]
    else:
        ef_ref = refs[0]
        refs = refs[1:]
    (Wn0, bn0, We0, Wvn0, Wg0, bg0, Wv0p,
     Whb1, Ws1, bs1, Wvb1, Wg1, bg1,
     Whb2, Ws2, bs2, Wvb2, Wg2, bg2,
     g1, b1,
     Whf0, Wsf0, bsf0, Wvf0, Wgf0, bgf0,
     Whf1, Wsf1, bsf1, Wvf1, Wgf1, bgf1,
     g2, b2) = refs[:35]
    outs = refs[35:]
    x_out = outs[0]
    ne = NB * 16
    X = x_ref[...]
    Xg = xg_ref[...]
    if first:
        es, eV16 = _edge_gvp(Xg, caf_b[...], cen, eWh, eWs, ebs, eWv, eWg,
                             ebg, NB=NB)
        efv = jnp.concatenate([es, eV16], axis=1)
        outs[1][...] = efv
    else:
        efv = ef_ref[...]
    s_d = X[:, 0:128]
    Vd48 = X[:, 128:176]

    def rep(t):
        return jnp.broadcast_to(t[:, None, :], (NB, 16, t.shape[1])
                                ).reshape(ne, t.shape[1])

    Mn = jnp.dot(X[:, 0:176], Wn0[...], preferred_element_type=F32) + bn0[...]
    ein = jnp.concatenate([Xg[:, 0:176], efv], axis=1)
    M = jnp.dot(ein, We0[...], preferred_element_type=F32) + rep(Mn)
    Vh = M[:, 128:248]
    vn = jnp.sqrt(Vh[:, 0:40] * Vh[:, 0:40] + Vh[:, 40:80] * Vh[:, 40:80]
                  + Vh[:, 80:120] * Vh[:, 80:120] + 1e-8)
    sm = M[:, 0:128] + jnp.dot(vn, Wvn0[...], preferred_element_type=F32)
    gate = jax.nn.sigmoid(jnp.dot(sm, Wg0[...], preferred_element_type=F32)
                          + bg0[...])
    ms = jax.nn.relu(sm)
    mV = (jnp.dot(Vh, Wv0p[...], preferred_element_type=F32)
          * jnp.concatenate([gate, gate, gate], axis=1))
    ms, mV = _gvp_packed(ms, mV, Whb1, Ws1, bs1, Wvb1, Wg1, bg1, True)
    ms, mV = _gvp_packed(ms, mV, Whb2, Ws2, bs2, Wvb2, Wg2, bg2, False)
    ags = jnp.sum(ms.reshape(NB, 16, 128), axis=1) * (1.0 / 16.0)
    agV = jnp.sum(mV.reshape(NB, 16, 48), axis=1) * (1.0 / 16.0)
    s_n = _ln(s_d + ags, g1, b1)
    V_n = _vln48(Vd48 + agV)
    fs, fV = _gvp_packed(s_n, V_n, Whf0, Wsf0, bsf0, Wvf0, Wgf0, bgf0, True)
    fs, fV = _gvp_packed(fs, fV, Whf1, Wsf1, bsf1, Wvf1, Wgf1, bgf1, False)
    s_o = _ln(s_n + fs, g2, b2)
    V_o = _vln48(V_n + fV)
    x_out[...] = jnp.concatenate([s_o, V_o, X[:, 176:XW]], axis=1)


def _layer(X, Xg, wlist, extra):
    N = X.shape[0]
    E = Xg.shape[0]
    NB = 256
    first = len(extra) > 1
    if first:
        especs = ([pl.BlockSpec((NB, 16), lambda i: (i, 0)),
                   pl.BlockSpec((1, 16), lambda i: (0, 0))]
                  + [pl.BlockSpec(w.shape, lambda i: (0,) * w.ndim)
                     for w in extra[2:]])
        out_specs = [pl.BlockSpec((NB, XW), lambda i: (i, 0)),
                     pl.BlockSpec((NB * 16, 48), lambda i: (i, 0))]
        out_shape = [jax.ShapeDtypeStruct((N, XW), F32),
                     jax.ShapeDtypeStruct((E, 48), F32)]
    else:
        especs = [pl.BlockSpec((NB * 16, 48), lambda i: (i, 0))]
        out_specs = [pl.BlockSpec((NB, XW), lambda i: (i, 0))]
        out_shape = [jax.ShapeDtypeStruct((N, XW), F32)]
    wspecs = [pl.BlockSpec(w.shape, lambda i: (0,) * w.ndim) for w in wlist]
    res = pl.pallas_call(
        functools.partial(_layer_body, first, NB),
        grid=(N // NB,),
        in_specs=[pl.BlockSpec((NB, XW), lambda i: (i, 0)),
                  pl.BlockSpec((NB * 16, XW), lambda i: (i, 0))]
        + especs + wspecs,
        out_specs=out_specs,
        out_shape=out_shape,
    )(X, Xg, *extra, *wlist)
    return res if first else (res[0], None)


def _unit(v, axis=-1, eps=1e-8):
    return v / jnp.sqrt(jnp.sum(v * v, axis=axis, keepdims=True) + eps)


def _dih_feats(coords):
    Bv, Lv = coords.shape[:2]
    Xf = coords.reshape(Bv, Lv * 3, 3)
    dX = Xf[:, 1:] - Xf[:, :-1]
    U = _unit(dX)
    u2, u1, u0 = U[:, :-2], U[:, 1:-1], U[:, 2:]
    n2 = _unit(jnp.cross(u2, u1))
    n1 = _unit(jnp.cross(u1, u0))
    cosD = jnp.clip(jnp.sum(n2 * n1, -1), -1 + 1e-7, 1 - 1e-7)
    D = jnp.sign(jnp.sum(u2 * n1, -1)) * jnp.arccos(cosD)
    D = jnp.pad(D, ((0, 0), (1, 2)))
    D = D.reshape(Bv, Lv, 3)
    return jnp.concatenate([jnp.cos(D), jnp.sin(D)], -1)


def _gvp_w(p):
    return dict(p, bs2=p["bs"].reshape(1, -1), bg2=p["bg"].reshape(1, -1))


def _bd3(A, pad_to=None):
    r, c = A.shape
    cp = c if pad_to is None else pad_to
    Z = jnp.zeros((3 * r, 3 * cp), F32)
    for k in range(3):
        Z = Z.at[k * r:(k + 1) * r, k * cp:k * cp + c].set(A)
    return Z


def _pack_small(q):
    return [_bd3(q["Wh"]), q["Ws"], q["bs2"], _bd3(q["Wv"]),
            q["Wg"], q["bg2"]]


def _pack_msg0(q):
    Wh, Ws, Wv = q["Wh"], q["Ws"], q["Wv"]
    Wn0 = jnp.zeros((176, 248), F32)
    Wn0 = Wn0.at[0:128, 0:128].set(Ws[0:128, :])
    Wn0 = Wn0.at[128:176, 128:248].set(_bd3(Wh[0:16, :], pad_to=40))
    bn0 = jnp.zeros((1, 248), F32).at[0, 0:128].set(q["bs"])
    We0 = jnp.zeros((224, 248), F32)
    We0 = We0.at[0:128, 0:128].set(Ws[128:256, :])
    We0 = We0.at[128:176, 128:248].set(_bd3(Wh[16:32, :], pad_to=40))
    We0 = We0.at[176:208, 0:128].set(Ws[256:288, :])
    for c in range(3):
        We0 = We0.at[208 + c, 128 + 40 * c:161 + 40 * c].set(Wh[32, :])
    Wvn0 = jnp.zeros((40, 128), F32).at[0:33, :].set(Ws[288:321, :])
    Wv0p = jnp.zeros((120, 48), F32)
    for c in range(3):
        Wv0p = Wv0p.at[40 * c:40 * c + 33, 16 * c:16 * c + 16].set(Wv)
    return [Wn0, bn0, We0, Wvn0, q["Wg"], q["bg2"], Wv0p]


def kernel(coords, coord_mask, padding_mask, params):
    Bv, Lv = coords.shape[:2]
    N = Bv * Lv
    K = 16
    CA = coords[:, :, 1, :]
    caf_pad = jnp.concatenate(
        [CA, jnp.zeros((Bv, Lv, 13), F32)], axis=-1)
    cat_pad = caf_pad.transpose(0, 2, 1)

    src = _topk(caf_pad, cat_pad, Lv, K).reshape(N * K)

    dih = _dih_feats(coords).reshape(N, 6)
    dih8 = jnp.concatenate([dih, jnp.zeros((N, 2), F32)], axis=1)
    fwdv = jnp.pad(_unit(CA[:, 1:] - CA[:, :-1]), ((0, 0), (0, 1), (0, 0)))
    bwdv = jnp.pad(_unit(CA[:, :-1] - CA[:, 1:]), ((0, 0), (1, 0), (0, 0)))
    ori = jnp.stack([fwdv, bwdv], axis=-1).reshape(N, 6)
    ori8 = jnp.concatenate([ori, jnp.zeros((N, 2), F32)], axis=1)
    caf_flat = caf_pad.reshape(N, 16)
    X = _node_embed(dih8, ori8, caf_flat, _gvp_w(params["node_embed"]))

    centers = jnp.linspace(0.0, 20.0, 16).reshape(1, 16).astype(F32)
    ep = _gvp_w(params["edge_embed"])
    ef = None
    for li, lp in enumerate(params["layers"]):
        m0, m1, m2 = (_gvp_w(q) for q in lp["msg"])
        f0, f1 = (_gvp_w(q) for q in lp["ff"])
        wlist = _pack_msg0(m0) + _pack_small(m1) + _pack_small(m2)
        wlist += [lp["ln1"]["g"].reshape(1, -1), lp["ln1"]["b"].reshape(1, -1)]
        wlist += _pack_small(f0) + _pack_small(f1)
        wlist += [lp["ln2"]["g"].reshape(1, -1), lp["ln2"]["b"].reshape(1, -1)]
        Xg = _sc_gather_rows(X, src.reshape(-1, 128))
        if li == 0:
            extra = (caf_flat, centers, ep["Wh"], ep["Ws"], ep["bs2"],
                     ep["Wv"], ep["Wg"].T, ep["bg2"])
        else:
            extra = (ef,)
        X, ef_new = _layer(X, Xg, wlist, extra)
        if li == 0:
            ef = ef_new

    s = X[:, 0:128].reshape(Bv, Lv, 128)
    V = jnp.stack([X[:, 128:144], X[:, 144:160], X[:, 160:176]],
                  axis=-1).reshape(Bv, Lv, 16, 3)
    return s, V
